# Initial kernel scaffold; baseline (speedup 1.0000x reference)
import jax
import jax.numpy as jnp
from jax.experimental import pallas as pl


def kernel(x, edge_index, edge_pairs, params):
    raise NotImplementedError("write your pallas kernel here")


# speedup vs baseline: 1.0001x; 1.0001x over previous
import jax
import jax.numpy as jnp
from jax.experimental import pallas as pl
from jax.experimental.pallas import tpu as pltpu

N = 10000
E = 320000
P = 512
OUT = 128
MAX_SPD = 5


def _head_body(hs_ref, ht_ref, bspd_ref, cn_ref, aa_ref, ja_ref,
               wq, wk, wv,
               cw1, cb1, cw2, cb2,
               aw1, ab1, aw2, ab2,
               jw1, jb1, jw2, jb2,
               mw1, mb1, mg, mbe, mw2, mb2, mw3, mb3,
               out_ref):
    hs = hs_ref[...]
    ht = ht_ref[...]
    q = jnp.dot(hs, wq[...], preferred_element_type=jnp.float32)
    k = jnp.dot(ht, wk[...], preferred_element_type=jnp.float32)
    v = jnp.dot(ht, wv[...], preferred_element_type=jnp.float32)
    a = (q * k).sum(-1, keepdims=True) * (1.0 / (OUT ** 0.5))

    def mlp(val, w1, b1, w2, b2):
        h = jnp.maximum(jnp.dot(val, w1[...], preferred_element_type=jnp.float32)
                        + b1[...], 0.0)
        return jnp.dot(h, w2[...], preferred_element_type=jnp.float32) + b2[...]

    s = (bspd_ref[...]
         + mlp(cn_ref[...], cw1, cb1, cw2, cb2)
         + mlp(aa_ref[...], aw1, ab1, aw2, ab2)
         + mlp(ja_ref[...], jw1, jb1, jw2, jb2))
    attn = jax.nn.sigmoid(a + s) * v
    z = jnp.concatenate([hs, ht, attn], axis=-1)
    z = jnp.dot(z, mw1[...], preferred_element_type=jnp.float32) + mb1[...]
    mu = z.mean(-1, keepdims=True)
    var = ((z - mu) ** 2).mean(-1, keepdims=True)
    z = (z - mu) / jnp.sqrt(var + 1e-5) * mg[...] + mbe[...]
    z = jnp.maximum(z, 0.0)
    z = jnp.maximum(jnp.dot(z, mw2[...], preferred_element_type=jnp.float32)
                    + mb2[...], 0.0)
    z = jnp.dot(z, mw3[...], preferred_element_type=jnp.float32) + mb3[...]
    out_ref[...] = z


def _head(hs, ht, bspd, cn, aa, ja, p):
    args = (hs, ht, bspd, cn[:, None], aa[:, None], ja[:, None],
            p['WQ'], p['WK'], p['WV'],
            p['cn_w1'], p['cn_b1'][None, :], p['cn_w2'], p['cn_b2'][None, :],
            p['aa_w1'], p['aa_b1'][None, :], p['aa_w2'], p['aa_b2'][None, :],
            p['ja_w1'], p['ja_b1'][None, :], p['ja_w2'], p['ja_b2'][None, :],
            p['m_w1'], p['m_b1'][None, :], p['m_g'][None, :], p['m_be'][None, :],
            p['m_w2'], p['m_b2'][None, :], p['m_w3'], p['m_b3'][None, :])
    out = pl.pallas_call(
        _head_body,
        out_shape=jax.ShapeDtypeStruct((P, 1), jnp.float32),
    )(*args)
    return out[:, 0]


def _gcn(x, w, b, src, dst, n):
    h = x @ w
    loop = jnp.arange(n, dtype=src.dtype)
    s = jnp.concatenate([src, loop])
    d = jnp.concatenate([dst, loop])
    deg = jnp.zeros((n,), jnp.float32).at[d].add(1.0)
    dis = 1.0 / jnp.sqrt(deg)
    norm = dis[s] * dis[d]
    out = jnp.zeros((n, h.shape[1]), jnp.float32).at[d].add(norm[:, None] * h[s])
    return out + b


def _ln(x, g, b):
    mu = x.mean(-1, keepdims=True)
    var = x.var(-1, keepdims=True)
    return (x - mu) / jnp.sqrt(var + 1e-5) * g + b


def kernel(x, edge_index, edge_pairs, params):
    p = params
    n = x.shape[0]
    src = edge_index[0].astype(jnp.int32)
    dst = edge_index[1].astype(jnp.int32)
    ps = edge_pairs[:, 0].astype(jnp.int32)
    pd = edge_pairs[:, 1].astype(jnp.int32)

    adj = jnp.zeros((n, n), jnp.float32)
    adj = adj.at[src, dst].max(1.0).at[dst, src].max(1.0)
    deg = adj.sum(-1)
    log_deg = jnp.maximum(jnp.log(jnp.maximum(deg, 1.0)), 1e-8)
    au = adj[ps]
    av = adj[pd]
    common = au * av
    cn = common.sum(-1)
    aa = (common / log_deg[None, :]).sum(-1)
    un = deg[ps] + deg[pd] - cn
    ja = jnp.where(un > 0, cn / jnp.where(un > 0, un, 1.0), 0.0)

    rows = jnp.arange(P)
    r = jnp.zeros((P, n), jnp.float32).at[rows, ps].set(1.0)
    hits = [r[rows, pd] > 0]
    for _ in range(MAX_SPD - 1):
        r = jnp.minimum(r + r @ adj, 1.0)
        hits.append(r[rows, pd] > 0)
    dist_code = (0, 1, 3, 4, 5)
    spd_idx = jnp.full((P,), MAX_SPD + 1, jnp.int32)
    for d in range(MAX_SPD - 1, -1, -1):
        spd_idx = jnp.where(hits[d], jnp.int32(dist_code[d]), spd_idx)
    bspd = p['spd_emb'][spd_idx]

    h = _ln(jax.nn.relu(_gcn(x, p['W1'], p['b1'], src, dst, n)), p['g1'], p['be1'])
    h = _ln(jax.nn.relu(_gcn(h, p['W2'], p['b2'], src, dst, n)), p['g2'], p['be2'])
    h = _ln(_gcn(h, p['W3'], p['b3'], src, dst, n) + x @ p['Wskip'], p['g3'], p['be3'])
    hs = h[ps]
    ht = h[pd]
    return _head(hs, ht, bspd, cn, aa, ja, p)


# speedup vs baseline: 3.3238x; 3.3235x over previous
import functools

import jax
import jax.numpy as jnp
from jax import lax
from jax.experimental import pallas as pl
from jax.experimental.pallas import tpu as pltpu
from jax.experimental.pallas import tpu_sc as plsc

N = 10000
NP = 10240
E = 320000
P = 512
OUT = 128
MAX_SPD = 5


_ZCH = 51200
_ZITERS = (NP * NP) // 16 // _ZCH
_K = 80
_EITERS = E // 16 // _K
_DEGP = 10240


def _build_body(esrc, edst, adj_flat, deg_out, zeros_v, ones_v, src_v, dst_v,
                linf_v, linr_v, deg_acc):
    s = lax.axis_index("s")

    def fill_z(i, _):
        zeros_v[pl.ds(i * 16, 16)] = jnp.zeros((16,), jnp.float32)
        return _
    lax.fori_loop(0, _ZCH // 16, fill_z, 0)
    for i in range(_K // 16):
        ones_v[pl.ds(i * 16, 16)] = jnp.ones((16,), jnp.float32)

    zbase = s * (_ZITERS * _ZCH)

    def zero_step(i, _):
        pltpu.sync_copy(zeros_v, adj_flat.at[pl.ds(zbase + i * _ZCH, _ZCH)])
        return _
    lax.fori_loop(0, _ZITERS, zero_step, 0)
    pltpu.sync_copy(zeros_v.at[pl.ds(0, _DEGP // 16)],
                    deg_acc.at[pl.ds(s * (_DEGP // 16), _DEGP // 16)])
    plsc.subcore_barrier()

    ebase = s * (_EITERS * _K)

    def edge_step(j, _):
        e0 = ebase + j * _K
        pltpu.sync_copy(esrc.at[pl.ds(e0, _K)], src_v)
        pltpu.sync_copy(edst.at[pl.ds(e0, _K)], dst_v)
        for t in range(_K // 16):
            sl = pl.ds(t * 16, 16)
            sv = src_v[sl]
            dv = dst_v[sl]
            linf_v[sl] = sv * NP + dv
            linr_v[sl] = dv * NP + sv
        pltpu.sync_copy(ones_v, adj_flat.at[linf_v])
        pltpu.sync_copy(ones_v, adj_flat.at[linr_v])
        pltpu.sync_copy(ones_v, deg_acc.at[dst_v], add=True)
        return _
    lax.fori_loop(0, _EITERS, edge_step, 0)
    plsc.subcore_barrier()

    dsl = pl.ds(s * (_DEGP // 16), _DEGP // 16)
    pltpu.sync_copy(deg_acc.at[dsl], deg_out.at[dsl])


def _sc_build(esrc, edst):
    mesh = plsc.VectorSubcoreMesh(core_axis_name="c", subcore_axis_name="s",
                                  num_cores=1)
    f = pl.kernel(
        _build_body,
        out_type=(jax.ShapeDtypeStruct((NP * NP,), jnp.float32),
                  jax.ShapeDtypeStruct((_DEGP,), jnp.float32)),
        mesh=mesh,
        scratch_types=(pltpu.VMEM((_ZCH,), jnp.float32),
                       pltpu.VMEM((_K,), jnp.float32),
                       pltpu.VMEM((_K,), jnp.int32),
                       pltpu.VMEM((_K,), jnp.int32),
                       pltpu.VMEM((_K,), jnp.int32),
                       pltpu.VMEM((_K,), jnp.int32),
                       pltpu.VMEM_SHARED((_DEGP,), jnp.float32)),
    )
    return f(esrc, edst)


_NA = 10240


def _agg_body(dh, esrc, edst, hlo, hhi, out, acc, zrow_v, src_v, dst_v, gbuf, sem):
    c = lax.axis_index("c")
    s = lax.axis_index("s")
    rows_per_tile = _NA // 16
    zr = 128

    def fill_z(i, _):
        r = i // (dh // 16)
        col = (i % (dh // 16)) * 16
        zrow_v[r, pl.ds(col, 16)] = jnp.zeros((16,), jnp.float32)
        return _
    lax.fori_loop(0, zr * (dh // 16), fill_z, 0)
    row0 = s * rows_per_tile
    for z in range(rows_per_tile // zr):
        pltpu.sync_copy(zrow_v, acc.at[pl.ds(row0 + z * zr, zr)])
    plsc.subcore_barrier()

    ebase = s * (_EITERS * _K)

    def edge_step(j, carry):
        e0 = ebase + j * _K
        pltpu.sync_copy(esrc.at[pl.ds(e0, _K)], src_v)
        pltpu.sync_copy(edst.at[pl.ds(e0, _K)], dst_v)

        @pl.when(c == 0)
        def _glo():
            pltpu.async_copy(hlo.at[src_v], gbuf, sem).wait()

        @pl.when(c == 1)
        def _ghi():
            pltpu.async_copy(hhi.at[src_v], gbuf, sem).wait()

        pltpu.sync_copy(gbuf, acc.at[dst_v], add=True)
        return carry
    lax.fori_loop(0, _EITERS, edge_step, 0)
    plsc.subcore_barrier()

    rsl = pl.ds(row0, rows_per_tile)
    pltpu.sync_copy(acc.at[rsl], out.at[c, rsl])


def _agg2_body(dh, esrc, edst, hf, out, acc, zrow_v, src_v, dst_v, gbuf, sem):
    c = lax.axis_index("c")
    s = lax.axis_index("s")
    rows_per_tile = _NA // 16
    zr = 128

    def fill_z(i, _):
        r = i // (dh // 16)
        col = (i % (dh // 16)) * 16
        zrow_v[r, pl.ds(col, 16)] = jnp.zeros((16,), jnp.float32)
        return _
    lax.fori_loop(0, zr * (dh // 16), fill_z, 0)
    row0 = s * rows_per_tile
    for z in range(rows_per_tile // zr):
        pltpu.sync_copy(zrow_v, acc.at[pl.ds(row0 + z * zr, zr)])
    plsc.subcore_barrier()

    wid = c * 16 + s
    niters = E // 32 // _K
    ebase = wid * (niters * _K)

    def edge_step(j, carry):
        e0 = ebase + j * _K
        pltpu.sync_copy(esrc.at[pl.ds(e0, _K)], src_v)
        pltpu.sync_copy(edst.at[pl.ds(e0, _K)], dst_v)
        pltpu.async_copy(hf.at[src_v], gbuf, sem).wait()
        pltpu.sync_copy(gbuf, acc.at[dst_v], add=True)
        return carry
    lax.fori_loop(0, niters, edge_step, 0)
    plsc.subcore_barrier()

    rsl = pl.ds(row0, rows_per_tile)
    pltpu.sync_copy(acc.at[rsl], out.at[c, rsl])


def _sc_aggregate_edges(esrc, edst, h_full):
    dh = h_full.shape[1]
    mesh = plsc.VectorSubcoreMesh(core_axis_name="c", subcore_axis_name="s")
    f = pl.kernel(
        functools.partial(_agg2_body, dh),
        out_type=jax.ShapeDtypeStruct((2, _NA, dh), jnp.float32),
        mesh=mesh,
        scratch_types=(pltpu.VMEM_SHARED((_NA, dh), jnp.float32),
                       pltpu.VMEM((128, dh), jnp.float32),
                       pltpu.VMEM((_K,), jnp.int32),
                       pltpu.VMEM((_K,), jnp.int32),
                       pltpu.VMEM((_K, dh), jnp.float32),
                       pltpu.SemaphoreType.DMA),
    )
    return f(esrc, edst, h_full)


def _sc_aggregate(esrc, edst, h_lo, h_hi):
    dh = h_lo.shape[1]
    mesh = plsc.VectorSubcoreMesh(core_axis_name="c", subcore_axis_name="s")
    f = pl.kernel(
        functools.partial(_agg_body, dh),
        out_type=jax.ShapeDtypeStruct((2, _NA, dh), jnp.float32),
        mesh=mesh,
        scratch_types=(pltpu.VMEM_SHARED((_NA, dh), jnp.float32),
                       pltpu.VMEM((128, dh), jnp.float32),
                       pltpu.VMEM((_K,), jnp.int32),
                       pltpu.VMEM((_K,), jnp.int32),
                       pltpu.VMEM((_K, dh), jnp.float32),
                       pltpu.SemaphoreType.DMA),
    )
    return f(esrc, edst, h_lo, h_hi)


_PREP_RB = 256
_PREP_STEPS = NP // _PREP_RB


def _prep_body(adj_ref, abf_ref, deg_ref, ldi_ref):
    i = pl.program_id(0)
    blk = adj_ref[...]
    abf_ref[...] = blk.astype(jnp.bfloat16)

    @pl.when(i == 0)
    def _():
        deg_ref[...] = jnp.zeros_like(deg_ref)
    deg_ref[...] += blk.sum(axis=0, keepdims=True)

    @pl.when(i == _PREP_STEPS - 1)
    def _():
        d = deg_ref[...]
        ldi_ref[...] = 1.0 / jnp.maximum(jnp.log(jnp.maximum(d, 1.0)), 1e-8)


def _tc_prep(adj):
    return pl.pallas_call(
        _prep_body,
        grid=(_PREP_STEPS,),
        in_specs=[pl.BlockSpec((_PREP_RB, NP), lambda i: (i, 0))],
        out_specs=[pl.BlockSpec((_PREP_RB, NP), lambda i: (i, 0)),
                   pl.BlockSpec((1, NP), lambda i: (0, 0)),
                   pl.BlockSpec((1, NP), lambda i: (0, 0))],
        out_shape=[jax.ShapeDtypeStruct((NP, NP), jnp.bfloat16),
                   jax.ShapeDtypeStruct((1, NP), jnp.float32),
                   jax.ShapeDtypeStruct((1, NP), jnp.float32)],
    )(adj)


def _pair_body(ps_ref, pd_ref, au_ref, av_ref, ldi_ref, r1_ref, feats_ref):
    i = pl.program_id(0)
    u = ps_ref[i]
    v = pd_ref[i]
    au = au_ref[0].astype(jnp.float32)
    av = av_ref[0].astype(jnp.float32)
    ldi = ldi_ref[...]
    prod = au * av
    iota = lax.broadcasted_iota(jnp.int32, (1, NP), 1)
    r1 = jnp.minimum(au + jnp.where(iota == u, 1.0, 0.0), 1.0)
    r1_ref[0] = r1.astype(jnp.bfloat16)
    hit1 = jnp.sum(jnp.where(iota == v, r1, 0.0))
    i8 = lax.broadcasted_iota(jnp.int32, (1, 8), 1)
    row = jnp.where(i8 == 0, prod.sum(), 0.0)
    row = jnp.where(i8 == 1, (prod * ldi).sum(), row)
    row = jnp.where(i8 == 2, au.sum(), row)
    row = jnp.where(i8 == 3, av.sum(), row)
    row = jnp.where(i8 == 4, hit1, row)
    feats_ref[0] = row


def _tc_pair(adj3, ldi, ps, pd):
    grid_spec = pltpu.PrefetchScalarGridSpec(
        num_scalar_prefetch=2,
        grid=(P,),
        in_specs=[
            pl.BlockSpec((1, 1, NP), lambda i, ps, pd: (ps[i], 0, 0)),
            pl.BlockSpec((1, 1, NP), lambda i, ps, pd: (pd[i], 0, 0)),
            pl.BlockSpec((1, NP), lambda i, ps, pd: (0, 0)),
        ],
        out_specs=[
            pl.BlockSpec((1, 1, NP), lambda i, ps, pd: (i, 0, 0)),
            pl.BlockSpec((1, 1, 8), lambda i, ps, pd: (i, 0, 0)),
        ],
    )
    return pl.pallas_call(
        _pair_body,
        grid_spec=grid_spec,
        out_shape=[jax.ShapeDtypeStruct((P, 1, NP), jnp.bfloat16),
                   jax.ShapeDtypeStruct((P, 1, 8), jnp.float32)],
    )(ps, pd, adj3, adj3, ldi)


_MMB = 1280
_MMG = NP // _MMB


def _mm_body(rk_ref, adj_ref, rj_ref, out_ref, acc_ref):
    k = pl.program_id(1)

    @pl.when(k == 0)
    def _():
        acc_ref[...] = jnp.zeros_like(acc_ref)
    acc_ref[...] += jnp.dot(rk_ref[...], adj_ref[...],
                            preferred_element_type=jnp.float32)

    @pl.when(k == _MMG - 1)
    def _():
        out_ref[...] = jnp.minimum(
            acc_ref[...] + rj_ref[...].astype(jnp.float32), 1.0
        ).astype(jnp.bfloat16)


def _tc_spd_step(r, adj_bf):
    return pl.pallas_call(
        _mm_body,
        grid=(_MMG, _MMG),
        in_specs=[pl.BlockSpec((P, _MMB), lambda j, k: (0, k)),
                  pl.BlockSpec((_MMB, _MMB), lambda j, k: (k, j)),
                  pl.BlockSpec((P, _MMB), lambda j, k: (0, j))],
        out_specs=pl.BlockSpec((P, _MMB), lambda j, k: (0, j)),
        out_shape=jax.ShapeDtypeStruct((P, NP), jnp.bfloat16),
        scratch_shapes=[pltpu.VMEM((P, _MMB), jnp.float32)],
    )(r, adj_bf, r)


def _extract_body(ps_ref, pd_ref, r2_ref, r3_ref, r4_ref, hits_ref):
    i = pl.program_id(0)
    v = pd_ref[i]
    iota = lax.broadcasted_iota(jnp.int32, (1, NP), 1)
    sel = jnp.where(iota == v, 1.0, 0.0)
    i8 = lax.broadcasted_iota(jnp.int32, (1, 8), 1)
    row = jnp.where(i8 == 0, (r2_ref[0].astype(jnp.float32) * sel).sum(), 0.0)
    row = jnp.where(i8 == 1, (r3_ref[0].astype(jnp.float32) * sel).sum(), row)
    row = jnp.where(i8 == 2, (r4_ref[0].astype(jnp.float32) * sel).sum(), row)
    hits_ref[0] = row


def _tc_extract(r2, r3, r4, ps, pd):
    grid_spec = pltpu.PrefetchScalarGridSpec(
        num_scalar_prefetch=2,
        grid=(P,),
        in_specs=[pl.BlockSpec((1, 1, NP), lambda i, ps, pd: (i, 0, 0))] * 3,
        out_specs=[pl.BlockSpec((1, 1, 8), lambda i, ps, pd: (i, 0, 0))],
    )
    return pl.pallas_call(
        _extract_body,
        grid_spec=grid_spec,
        out_shape=[jax.ShapeDtypeStruct((P, 1, 8), jnp.float32)],
    )(ps, pd, r2.reshape(P, 1, NP), r3.reshape(P, 1, NP),
      r4.reshape(P, 1, NP))[0]


_GRB = 1000
_GSTEPS = N // _GRB


def _scale_body(h_ref, w_ref, deg_ref, *out_refs):
    dis = lax.rsqrt(deg_ref[...] + 1.0)
    hw = jnp.dot(h_ref[...], w_ref[...], preferred_element_type=jnp.float32)
    hw = hw * dis
    if len(out_refs) == 2:
        half = hw.shape[1] // 2
        out_refs[0][...] = hw[:, :half]
        out_refs[1][...] = hw[:, half:]
    else:
        out_refs[0][...] = hw


def _tc_scale(h, w, deg_col, split=True):
    cin = h.shape[1]
    cout = w.shape[1]
    if split:
        half = cout // 2
        outs = [jax.ShapeDtypeStruct((N, half), jnp.float32),
                jax.ShapeDtypeStruct((N, half), jnp.float32)]
        ospecs = [pl.BlockSpec((_GRB, half), lambda i: (i, 0)),
                  pl.BlockSpec((_GRB, half), lambda i: (i, 0))]
    else:
        outs = [jax.ShapeDtypeStruct((N, cout), jnp.float32)]
        ospecs = [pl.BlockSpec((_GRB, cout), lambda i: (i, 0))]
    return pl.pallas_call(
        _scale_body,
        grid=(_GSTEPS,),
        in_specs=[pl.BlockSpec((_GRB, cin), lambda i: (i, 0)),
                  pl.BlockSpec((cin, cout), lambda i: (0, 0)),
                  pl.BlockSpec((_GRB, 1), lambda i: (i, 0))],
        out_specs=ospecs,
        out_shape=outs,
    )(h, w, deg_col)


def _post_body(relu, aglo_ref, aghi_ref, hlo_ref, hhi_ref, deg_ref,
               b_ref, g_ref, be_ref, out_ref):
    dis = lax.rsqrt(deg_ref[...] + 1.0)
    lo = aglo_ref[0] + hlo_ref[...]
    hi = aghi_ref[0] + hhi_ref[...]
    z = jnp.concatenate([lo, hi], axis=-1) * dis + b_ref[...]
    if relu:
        z = jnp.maximum(z, 0.0)
    mu = z.mean(-1, keepdims=True)
    var = ((z - mu) ** 2).mean(-1, keepdims=True)
    out_ref[...] = (z - mu) / jnp.sqrt(var + 1e-5) * g_ref[...] + be_ref[...]


def _tc_post(agg, h_lo, h_hi, deg_col, b, g, be, relu):
    dh = h_lo.shape[1]
    d = 2 * dh
    return pl.pallas_call(
        functools.partial(_post_body, relu),
        grid=(_GSTEPS,),
        in_specs=[pl.BlockSpec((1, _GRB, dh), lambda i: (0, i, 0)),
                  pl.BlockSpec((1, _GRB, dh), lambda i: (1, i, 0)),
                  pl.BlockSpec((_GRB, dh), lambda i: (i, 0)),
                  pl.BlockSpec((_GRB, dh), lambda i: (i, 0)),
                  pl.BlockSpec((_GRB, 1), lambda i: (i, 0)),
                  pl.BlockSpec((1, d), lambda i: (0, 0)),
                  pl.BlockSpec((1, d), lambda i: (0, 0)),
                  pl.BlockSpec((1, d), lambda i: (0, 0))],
        out_specs=[pl.BlockSpec((_GRB, d), lambda i: (i, 0))],
        out_shape=[jax.ShapeDtypeStruct((N, d), jnp.float32)],
    )(agg, agg, h_lo, h_hi, deg_col, b, g, be)[0]


def _post3_body(ag0_ref, ag1_ref, hf_ref, deg_ref, x_ref, wsk_ref,
                b_ref, g_ref, be_ref, out_ref):
    dis = lax.rsqrt(deg_ref[...] + 1.0)
    full = ag0_ref[0] + ag1_ref[0] + hf_ref[...]
    skip = jnp.dot(x_ref[...], wsk_ref[...], preferred_element_type=jnp.float32)
    z = full * dis + b_ref[...] + skip
    mu = z.mean(-1, keepdims=True)
    var = ((z - mu) ** 2).mean(-1, keepdims=True)
    out_ref[...] = (z - mu) / jnp.sqrt(var + 1e-5) * g_ref[...] + be_ref[...]


def _tc_post3(agg, h_full, deg_col, x, wsk, b, g, be):
    d = h_full.shape[1]
    cin = x.shape[1]
    return pl.pallas_call(
        _post3_body,
        grid=(_GSTEPS,),
        in_specs=[pl.BlockSpec((1, _GRB, d), lambda i: (0, i, 0)),
                  pl.BlockSpec((1, _GRB, d), lambda i: (1, i, 0)),
                  pl.BlockSpec((_GRB, d), lambda i: (i, 0)),
                  pl.BlockSpec((_GRB, 1), lambda i: (i, 0)),
                  pl.BlockSpec((_GRB, cin), lambda i: (i, 0)),
                  pl.BlockSpec((cin, d), lambda i: (0, 0)),
                  pl.BlockSpec((1, d), lambda i: (0, 0)),
                  pl.BlockSpec((1, d), lambda i: (0, 0)),
                  pl.BlockSpec((1, d), lambda i: (0, 0))],
        out_specs=[pl.BlockSpec((_GRB, d), lambda i: (i, 0))],
        out_shape=[jax.ShapeDtypeStruct((N, d), jnp.float32)],
    )(agg, agg, h_full, deg_col, x, wsk, b, g, be)[0]


def _gather_body(ps_ref, pd_ref, hu_ref, hv_ref, hs_ref, ht_ref):
    hs_ref[...] = hu_ref[...]
    ht_ref[...] = hv_ref[...]


def _tc_gather_pairs(h3, ps, pd):
    d = h3.shape[2]
    grid_spec = pltpu.PrefetchScalarGridSpec(
        num_scalar_prefetch=2,
        grid=(P,),
        in_specs=[pl.BlockSpec((1, 1, d), lambda i, ps, pd: (ps[i], 0, 0)),
                  pl.BlockSpec((1, 1, d), lambda i, ps, pd: (pd[i], 0, 0))],
        out_specs=[pl.BlockSpec((1, 1, d), lambda i, ps, pd: (i, 0, 0)),
                   pl.BlockSpec((1, 1, d), lambda i, ps, pd: (i, 0, 0))],
    )
    return pl.pallas_call(
        _gather_body,
        grid_spec=grid_spec,
        out_shape=[jax.ShapeDtypeStruct((P, 1, d), jnp.float32),
                   jax.ShapeDtypeStruct((P, 1, d), jnp.float32)],
    )(ps, pd, h3, h3)


def _head_body(hs_ref, ht_ref, feats_ref, hits_ref, ep_ref, emb_ref,
               wq, wk, wv,
               cw1, cb1, cw2, cb2,
               aw1, ab1, aw2, ab2,
               jw1, jb1, jw2, jb2,
               mw1, mb1, mg, mbe, mw2, mb2, mw3, mb3,
               out_ref):
    hs = hs_ref[...]
    ht = ht_ref[...]
    feats = feats_ref[...]
    hits = hits_ref[...]
    cn = feats[:, 0:1]
    aa = feats[:, 1:2]
    du = feats[:, 2:3]
    dv = feats[:, 3:4]
    h1 = feats[:, 4:5]
    un = du + dv - cn
    ja = jnp.where(un > 0, cn / jnp.where(un > 0, un, 1.0), 0.0)

    ps = ep_ref[:, 0:1]
    pd = ep_ref[:, 1:2]
    spd = jnp.full_like(ps, MAX_SPD + 1)
    spd = jnp.where(hits[:, 2:3] > 0, 5, spd)
    spd = jnp.where(hits[:, 1:2] > 0, 4, spd)
    spd = jnp.where(hits[:, 0:1] > 0, 3, spd)
    spd = jnp.where(h1 > 0, 1, spd)
    spd = jnp.where(ps == pd, 0, spd)
    emb = emb_ref[...]
    bspd = jnp.zeros_like(cn)
    for k in range(MAX_SPD + 2):
        bspd = jnp.where(spd == k, emb[0, k], bspd)

    q = jnp.dot(hs, wq[...], preferred_element_type=jnp.float32)
    kk = jnp.dot(ht, wk[...], preferred_element_type=jnp.float32)
    v = jnp.dot(ht, wv[...], preferred_element_type=jnp.float32)
    a = (q * kk).sum(-1, keepdims=True) * (1.0 / (OUT ** 0.5))

    def mlp(val, w1, b1, w2, b2):
        h = jnp.maximum(jnp.dot(val, w1[...], preferred_element_type=jnp.float32)
                        + b1[...], 0.0)
        return jnp.dot(h, w2[...], preferred_element_type=jnp.float32) + b2[...]

    s = (bspd
         + mlp(cn, cw1, cb1, cw2, cb2)
         + mlp(aa, aw1, ab1, aw2, ab2)
         + mlp(ja, jw1, jb1, jw2, jb2))
    attn = jax.nn.sigmoid(a + s) * v
    z = jnp.concatenate([hs, ht, attn], axis=-1)
    z = jnp.dot(z, mw1[...], preferred_element_type=jnp.float32) + mb1[...]
    mu = z.mean(-1, keepdims=True)
    var = ((z - mu) ** 2).mean(-1, keepdims=True)
    z = (z - mu) / jnp.sqrt(var + 1e-5) * mg[...] + mbe[...]
    z = jnp.maximum(z, 0.0)
    z = jnp.maximum(jnp.dot(z, mw2[...], preferred_element_type=jnp.float32)
                    + mb2[...], 0.0)
    z = jnp.dot(z, mw3[...], preferred_element_type=jnp.float32) + mb3[...]
    out_ref[...] = z


def _head(hs, ht, feats, hits, ep, p):
    emb_row = jnp.zeros((1, 8), jnp.float32).at[0, :MAX_SPD + 2].set(
        p['spd_emb'][:, 0])
    args = (hs, ht, feats, hits, ep, emb_row,
            p['WQ'], p['WK'], p['WV'],
            p['cn_w1'], p['cn_b1'][None, :], p['cn_w2'], p['cn_b2'][None, :],
            p['aa_w1'], p['aa_b1'][None, :], p['aa_w2'], p['aa_b2'][None, :],
            p['ja_w1'], p['ja_b1'][None, :], p['ja_w2'], p['ja_b2'][None, :],
            p['m_w1'], p['m_b1'][None, :], p['m_g'][None, :], p['m_be'][None, :],
            p['m_w2'], p['m_b2'][None, :], p['m_w3'], p['m_b3'][None, :])
    out = pl.pallas_call(
        _head_body,
        out_shape=jax.ShapeDtypeStruct((P, 1), jnp.float32),
    )(*args)
    return out[:, 0]


def kernel(x, edge_index, edge_pairs, params):
    p = params
    ei = edge_index.astype(jnp.int32)
    esrc = ei[0]
    edst = ei[1]
    ep = edge_pairs.astype(jnp.int32)
    ps = ep[:, 0]
    pd = ep[:, 1]

    adj_flat, deg_hist = _sc_build(esrc, edst)
    adj = adj_flat.reshape(NP, NP)
    deg_col = deg_hist[:N].reshape(N, 1)

    adj_bf, _, ldi = _tc_prep(adj)
    adj3 = adj_bf.reshape(NP, 1, NP)
    r1_3, feats3 = _tc_pair(adj3, ldi, ps, pd)
    r1 = r1_3.reshape(P, NP)
    r2 = _tc_spd_step(r1, adj_bf)
    r3 = _tc_spd_step(r2, adj_bf)
    r4 = _tc_spd_step(r3, adj_bf)
    hits = _tc_extract(r2, r3, r4, ps, pd).reshape(P, 8)
    feats = feats3.reshape(P, 8)

    lo, hi = _tc_scale(x, p['W1'], deg_col)
    agg = _sc_aggregate(esrc, edst, lo, hi)
    h = _tc_post(agg, lo, hi, deg_col, p['b1'][None, :], p['g1'][None, :],
                 p['be1'][None, :], relu=True)
    lo, hi = _tc_scale(h, p['W2'], deg_col)
    agg = _sc_aggregate(esrc, edst, lo, hi)
    h = _tc_post(agg, lo, hi, deg_col, p['b2'][None, :], p['g2'][None, :],
                 p['be2'][None, :], relu=True)
    hf = _tc_scale(h, p['W3'], deg_col, split=False)[0]
    agg = _sc_aggregate_edges(esrc, edst, hf)
    h = _tc_post3(agg, hf, deg_col, x, p['Wskip'], p['b3'][None, :],
                  p['g3'][None, :], p['be3'][None, :])

    hs3, ht3 = _tc_gather_pairs(h.reshape(N, 1, OUT), ps, pd)
    return _head(hs3.reshape(P, OUT), ht3.reshape(P, OUT), feats, hits, ep, p)


# speedup vs baseline: 3.9166x; 1.1783x over previous
import functools

import jax
import jax.numpy as jnp
from jax import lax
from jax.experimental import pallas as pl
from jax.experimental.pallas import tpu as pltpu
from jax.experimental.pallas import tpu_sc as plsc

N = 10000
NP = 10240
E = 320000
P = 512
OUT = 128
MAX_SPD = 5


_ZCH = 51200
_ZITERS = (NP * NP) // 16 // _ZCH
_K = 80
_EITERS = E // 16 // _K
_DEGP = 10240


def _build_body(esrc, edst, adj_flat, deg_out, zeros_v, ones_v,
                sst, dstt, linf0, linr0, linf1, linr1, deg_acc,
                semz, semi, semf0, semr0, semd0, semf1, semr1, semd1):
    s = lax.axis_index("s")

    def fill_z(i, _):
        zeros_v[pl.ds(i * 16, 16)] = jnp.zeros((16,), jnp.float32)
        return _
    lax.fori_loop(0, _ZCH // 16, fill_z, 0)
    for i in range(_K // 16):
        ones_v[pl.ds(i * 16, 16)] = jnp.ones((16,), jnp.float32)

    zbase = s * (_ZITERS * _ZCH)

    def zero_blk(i, _):
        for q in range(8):
            pltpu.async_copy(
                zeros_v, adj_flat.at[pl.ds(zbase + (i * 8 + q) * _ZCH, _ZCH)],
                semz)
        for q in range(8):
            pltpu.make_async_copy(
                zeros_v, adj_flat.at[pl.ds(zbase, _ZCH)], semz).wait()
        return _
    lax.fori_loop(0, _ZITERS // 8, zero_blk, 0)
    pltpu.sync_copy(zeros_v.at[pl.ds(0, _DEGP // 16)],
                    deg_acc.at[pl.ds(s * (_DEGP // 16), _DEGP // 16)])
    plsc.subcore_barrier()

    chb = 10
    nblk = (E // 16) // (chb * _K)
    ebase = s * (E // 16)
    linf = (linf0, linf1)
    linr = (linr0, linr1)
    semf = (semf0, semf1)
    semr = (semr0, semr1)
    semd = (semd0, semd1)

    def drain_slot(sl):
        pltpu.make_async_copy(ones_v, adj_flat.at[pl.ds(0, _K)], semf[sl]).wait()
        pltpu.make_async_copy(ones_v, adj_flat.at[pl.ds(0, _K)], semr[sl]).wait()
        pltpu.make_async_copy(ones_v, deg_acc.at[pl.ds(0, _K)], semd[sl]).wait()

    def blk(b, carry):
        eb = ebase + b * (chb * _K)

        @pl.when(b > 0)
        def _drain():
            drain_slot(0)
            drain_slot(1)

        for p in range(chb):
            pltpu.async_copy(esrc.at[pl.ds(eb + p * _K, _K)], sst.at[p], semi)
            pltpu.async_copy(edst.at[pl.ds(eb + p * _K, _K)], dstt.at[p], semi)
        for p in range(2 * chb):
            pltpu.make_async_copy(esrc.at[pl.ds(0, _K)], sst.at[0], semi).wait()
        for p in range(chb):
            sl = p % 2
            if p >= 2:
                drain_slot(sl)
            for t in range(_K // 16):
                tsl = pl.ds(t * 16, 16)
                sv = sst[p, tsl]
                dv = dstt[p, tsl]
                linf[sl][tsl] = sv * NP + dv
                linr[sl][tsl] = dv * NP + sv
            pltpu.async_copy(ones_v, adj_flat.at[linf[sl]], semf[sl])
            pltpu.async_copy(ones_v, adj_flat.at[linr[sl]], semr[sl])
            pltpu.async_copy(ones_v, deg_acc.at[dstt.at[p]], semd[sl], add=True)
        return carry
    lax.fori_loop(0, nblk, blk, 0)
    drain_slot(0)
    drain_slot(1)
    plsc.subcore_barrier()

    dsl = pl.ds(s * (_DEGP // 16), _DEGP // 16)
    pltpu.sync_copy(deg_acc.at[dsl], deg_out.at[dsl])


def _sc_build(esrc, edst):
    mesh = plsc.VectorSubcoreMesh(core_axis_name="c", subcore_axis_name="s",
                                  num_cores=1)
    f = pl.kernel(
        _build_body,
        out_type=(jax.ShapeDtypeStruct((NP * NP,), jnp.float32),
                  jax.ShapeDtypeStruct((_DEGP,), jnp.float32)),
        mesh=mesh,
        scratch_types=(pltpu.VMEM((_ZCH,), jnp.float32),
                       pltpu.VMEM((_K,), jnp.float32),
                       pltpu.VMEM((10, _K), jnp.int32),
                       pltpu.VMEM((10, _K), jnp.int32),
                       pltpu.VMEM((_K,), jnp.int32),
                       pltpu.VMEM((_K,), jnp.int32),
                       pltpu.VMEM((_K,), jnp.int32),
                       pltpu.VMEM((_K,), jnp.int32),
                       pltpu.VMEM_SHARED((_DEGP,), jnp.float32),
                       pltpu.SemaphoreType.DMA,
                       pltpu.SemaphoreType.DMA,
                       pltpu.SemaphoreType.DMA,
                       pltpu.SemaphoreType.DMA,
                       pltpu.SemaphoreType.DMA,
                       pltpu.SemaphoreType.DMA,
                       pltpu.SemaphoreType.DMA,
                       pltpu.SemaphoreType.DMA),
    )
    return f(esrc, edst)


_NA = 10240


def _agg_body(dh, split, esrc, edst, tlo, thi, out, acc, zrow_v, sst, dstt,
              gb0, gb1, semi, semg0, semg1, sems0, sems1):
    c = lax.axis_index("c")
    s = lax.axis_index("s")
    rows_per_tile = _NA // 16

    def fill_z(i, _):
        r = i // (dh // 16)
        col = (i % (dh // 16)) * 16
        zrow_v[r, pl.ds(col, 16)] = jnp.zeros((16,), jnp.float32)
        return _
    lax.fori_loop(0, 128 * (dh // 16), fill_z, 0)
    row0 = s * rows_per_tile
    for z in range(rows_per_tile // 128):
        pltpu.async_copy(zrow_v, acc.at[pl.ds(row0 + z * 128, 128)], semi)
    for z in range(rows_per_tile // 128):
        pltpu.make_async_copy(zrow_v, acc.at[pl.ds(row0, 128)], semi).wait()
    plsc.subcore_barrier()

    chb = 10 if split else 5
    ept = (E // 16) if split else (E // 32)
    nblk = ept // (chb * _K)
    wid = s if split else (c * 16 + s)
    ebase = wid * ept
    gbufs = (gb0, gb1)
    semgs = (semg0, semg1)
    semss = (sems0, sems1)

    def drain_s(sl):
        pltpu.make_async_copy(gbufs[sl], acc.at[pl.ds(0, _K)], semss[sl]).wait()

    def wait_g(sl):
        pltpu.make_async_copy(tlo.at[pl.ds(0, _K)], gbufs[sl], semgs[sl]).wait()

    def blk(b, carry):
        eb = ebase + b * (chb * _K)

        @pl.when(b > 0)
        def _drain():
            drain_s(0)
            drain_s(1)

        for p in range(chb):
            pltpu.async_copy(esrc.at[pl.ds(eb + p * _K, _K)], sst.at[p], semi)
            pltpu.async_copy(edst.at[pl.ds(eb + p * _K, _K)], dstt.at[p], semi)
        for p in range(2 * chb):
            pltpu.make_async_copy(esrc.at[pl.ds(0, _K)], sst.at[0], semi).wait()
        for p in range(chb):
            sl = p % 2
            if p >= 2:
                drain_s(sl)
            if split:
                @pl.when(c == 0)
                def _glo():
                    pltpu.async_copy(tlo.at[sst.at[p]], gbufs[sl], semgs[sl])

                @pl.when(c == 1)
                def _ghi():
                    pltpu.async_copy(thi.at[sst.at[p]], gbufs[sl], semgs[sl])
            else:
                pltpu.async_copy(tlo.at[sst.at[p]], gbufs[sl], semgs[sl])
            if p >= 1:
                osl = 1 - sl
                wait_g(osl)
                pltpu.async_copy(gbufs[osl], acc.at[dstt.at[p - 1]], semss[osl],
                                 add=True)
        lsl = (chb - 1) % 2
        wait_g(lsl)
        pltpu.async_copy(gbufs[lsl], acc.at[dstt.at[chb - 1]], semss[lsl],
                         add=True)
        return carry
    lax.fori_loop(0, nblk, blk, 0)
    drain_s(0)
    drain_s(1)
    plsc.subcore_barrier()

    rsl = pl.ds(row0, rows_per_tile)
    pltpu.sync_copy(acc.at[rsl], out.at[c, rsl])


def _sc_aggregate(esrc, edst, h_lo, h_hi, split=True):
    dh = h_lo.shape[1]
    chb = 10 if split else 5
    mesh = plsc.VectorSubcoreMesh(core_axis_name="c", subcore_axis_name="s")
    f = pl.kernel(
        functools.partial(_agg_body, dh, split),
        out_type=jax.ShapeDtypeStruct((2, _NA, dh), jnp.float32),
        mesh=mesh,
        scratch_types=(pltpu.VMEM_SHARED((_NA, dh), jnp.float32),
                       pltpu.VMEM((128, dh), jnp.float32),
                       pltpu.VMEM((chb, _K), jnp.int32),
                       pltpu.VMEM((chb, _K), jnp.int32),
                       pltpu.VMEM((_K, dh), jnp.float32),
                       pltpu.VMEM((_K, dh), jnp.float32),
                       pltpu.SemaphoreType.DMA,
                       pltpu.SemaphoreType.DMA,
                       pltpu.SemaphoreType.DMA,
                       pltpu.SemaphoreType.DMA,
                       pltpu.SemaphoreType.DMA),
    )
    return f(esrc, edst, h_lo, h_hi)


def _sc_aggregate_edges(esrc, edst, h_full):
    return _sc_aggregate(esrc, edst, h_full, h_full, split=False)


_PREP_RB = 256
_PREP_STEPS = NP // _PREP_RB


def _prep_body(adj_ref, abf_ref, deg_ref, ldi_ref):
    i = pl.program_id(0)
    blk = adj_ref[...]
    abf_ref[...] = blk.astype(jnp.bfloat16)

    @pl.when(i == 0)
    def _():
        deg_ref[...] = jnp.zeros_like(deg_ref)
    deg_ref[...] += blk.sum(axis=0, keepdims=True)

    @pl.when(i == _PREP_STEPS - 1)
    def _():
        d = deg_ref[...]
        ldi_ref[...] = 1.0 / jnp.maximum(jnp.log(jnp.maximum(d, 1.0)), 1e-8)


def _tc_prep(adj):
    return pl.pallas_call(
        _prep_body,
        grid=(_PREP_STEPS,),
        in_specs=[pl.BlockSpec((_PREP_RB, NP), lambda i: (i, 0))],
        out_specs=[pl.BlockSpec((_PREP_RB, NP), lambda i: (i, 0)),
                   pl.BlockSpec((1, NP), lambda i: (0, 0)),
                   pl.BlockSpec((1, NP), lambda i: (0, 0))],
        out_shape=[jax.ShapeDtypeStruct((NP, NP), jnp.bfloat16),
                   jax.ShapeDtypeStruct((1, NP), jnp.float32),
                   jax.ShapeDtypeStruct((1, NP), jnp.float32)],
    )(adj)


def _pair_body(ps_ref, pd_ref, au_ref, av_ref, ldi_ref, r1_ref, feats_ref):
    i = pl.program_id(0)
    u = ps_ref[i]
    v = pd_ref[i]
    au = au_ref[0].astype(jnp.float32)
    av = av_ref[0].astype(jnp.float32)
    ldi = ldi_ref[...]
    prod = au * av
    iota = lax.broadcasted_iota(jnp.int32, (1, NP), 1)
    r1 = jnp.minimum(au + jnp.where(iota == u, 1.0, 0.0), 1.0)
    r1_ref[0] = r1.astype(jnp.bfloat16)
    hit1 = jnp.sum(jnp.where(iota == v, r1, 0.0))
    i8 = lax.broadcasted_iota(jnp.int32, (1, 8), 1)
    row = jnp.where(i8 == 0, prod.sum(), 0.0)
    row = jnp.where(i8 == 1, (prod * ldi).sum(), row)
    row = jnp.where(i8 == 2, au.sum(), row)
    row = jnp.where(i8 == 3, av.sum(), row)
    row = jnp.where(i8 == 4, hit1, row)
    feats_ref[0] = row


def _tc_pair(adj3, ldi, ps, pd):
    grid_spec = pltpu.PrefetchScalarGridSpec(
        num_scalar_prefetch=2,
        grid=(P,),
        in_specs=[
            pl.BlockSpec((1, 1, NP), lambda i, ps, pd: (ps[i], 0, 0)),
            pl.BlockSpec((1, 1, NP), lambda i, ps, pd: (pd[i], 0, 0)),
            pl.BlockSpec((1, NP), lambda i, ps, pd: (0, 0)),
        ],
        out_specs=[
            pl.BlockSpec((1, 1, NP), lambda i, ps, pd: (i, 0, 0)),
            pl.BlockSpec((1, 1, 8), lambda i, ps, pd: (i, 0, 0)),
        ],
    )
    return pl.pallas_call(
        _pair_body,
        grid_spec=grid_spec,
        out_shape=[jax.ShapeDtypeStruct((P, 1, NP), jnp.bfloat16),
                   jax.ShapeDtypeStruct((P, 1, 8), jnp.float32)],
    )(ps, pd, adj3, adj3, ldi)


_MMB = 1280
_MMG = NP // _MMB


def _mm_body(rk_ref, adj_ref, rj_ref, out_ref, acc_ref):
    k = pl.program_id(1)

    @pl.when(k == 0)
    def _():
        acc_ref[...] = jnp.zeros_like(acc_ref)
    acc_ref[...] += jnp.dot(rk_ref[...], adj_ref[...],
                            preferred_element_type=jnp.float32)

    @pl.when(k == _MMG - 1)
    def _():
        out_ref[...] = jnp.minimum(
            acc_ref[...] + rj_ref[...].astype(jnp.float32), 1.0
        ).astype(jnp.bfloat16)


def _tc_spd_step(r, adj_bf):
    return pl.pallas_call(
        _mm_body,
        grid=(_MMG, _MMG),
        in_specs=[pl.BlockSpec((P, _MMB), lambda j, k: (0, k)),
                  pl.BlockSpec((_MMB, _MMB), lambda j, k: (k, j)),
                  pl.BlockSpec((P, _MMB), lambda j, k: (0, j))],
        out_specs=pl.BlockSpec((P, _MMB), lambda j, k: (0, j)),
        out_shape=jax.ShapeDtypeStruct((P, NP), jnp.bfloat16),
        scratch_shapes=[pltpu.VMEM((P, _MMB), jnp.float32)],
    )(r, adj_bf, r)


def _extract_body(ps_ref, pd_ref, r2_ref, r3_ref, r4_ref, hits_ref):
    i = pl.program_id(0)
    v = pd_ref[i]
    iota = lax.broadcasted_iota(jnp.int32, (1, NP), 1)
    sel = jnp.where(iota == v, 1.0, 0.0)
    i8 = lax.broadcasted_iota(jnp.int32, (1, 8), 1)
    row = jnp.where(i8 == 0, (r2_ref[0].astype(jnp.float32) * sel).sum(), 0.0)
    row = jnp.where(i8 == 1, (r3_ref[0].astype(jnp.float32) * sel).sum(), row)
    row = jnp.where(i8 == 2, (r4_ref[0].astype(jnp.float32) * sel).sum(), row)
    hits_ref[0] = row


def _tc_extract(r2, r3, r4, ps, pd):
    grid_spec = pltpu.PrefetchScalarGridSpec(
        num_scalar_prefetch=2,
        grid=(P,),
        in_specs=[pl.BlockSpec((1, 1, NP), lambda i, ps, pd: (i, 0, 0))] * 3,
        out_specs=[pl.BlockSpec((1, 1, 8), lambda i, ps, pd: (i, 0, 0))],
    )
    return pl.pallas_call(
        _extract_body,
        grid_spec=grid_spec,
        out_shape=[jax.ShapeDtypeStruct((P, 1, 8), jnp.float32)],
    )(ps, pd, r2.reshape(P, 1, NP), r3.reshape(P, 1, NP),
      r4.reshape(P, 1, NP))[0]


_GRB = 1000
_GSTEPS = N // _GRB


def _scale_body(h_ref, w_ref, deg_ref, *out_refs):
    dis = lax.rsqrt(deg_ref[...] + 1.0)
    hw = jnp.dot(h_ref[...], w_ref[...], preferred_element_type=jnp.float32)
    hw = hw * dis
    if len(out_refs) == 2:
        half = hw.shape[1] // 2
        out_refs[0][...] = hw[:, :half]
        out_refs[1][...] = hw[:, half:]
    else:
        out_refs[0][...] = hw


def _tc_scale(h, w, deg_col, split=True):
    cin = h.shape[1]
    cout = w.shape[1]
    if split:
        half = cout // 2
        outs = [jax.ShapeDtypeStruct((N, half), jnp.float32),
                jax.ShapeDtypeStruct((N, half), jnp.float32)]
        ospecs = [pl.BlockSpec((_GRB, half), lambda i: (i, 0)),
                  pl.BlockSpec((_GRB, half), lambda i: (i, 0))]
    else:
        outs = [jax.ShapeDtypeStruct((N, cout), jnp.float32)]
        ospecs = [pl.BlockSpec((_GRB, cout), lambda i: (i, 0))]
    return pl.pallas_call(
        _scale_body,
        grid=(_GSTEPS,),
        in_specs=[pl.BlockSpec((_GRB, cin), lambda i: (i, 0)),
                  pl.BlockSpec((cin, cout), lambda i: (0, 0)),
                  pl.BlockSpec((_GRB, 1), lambda i: (i, 0))],
        out_specs=ospecs,
        out_shape=outs,
    )(h, w, deg_col)


def _post_body(relu, aglo_ref, aghi_ref, hlo_ref, hhi_ref, deg_ref,
               b_ref, g_ref, be_ref, out_ref):
    dis = lax.rsqrt(deg_ref[...] + 1.0)
    lo = aglo_ref[0] + hlo_ref[...]
    hi = aghi_ref[0] + hhi_ref[...]
    z = jnp.concatenate([lo, hi], axis=-1) * dis + b_ref[...]
    if relu:
        z = jnp.maximum(z, 0.0)
    mu = z.mean(-1, keepdims=True)
    var = ((z - mu) ** 2).mean(-1, keepdims=True)
    out_ref[...] = (z - mu) / jnp.sqrt(var + 1e-5) * g_ref[...] + be_ref[...]


def _tc_post(agg, h_lo, h_hi, deg_col, b, g, be, relu):
    dh = h_lo.shape[1]
    d = 2 * dh
    return pl.pallas_call(
        functools.partial(_post_body, relu),
        grid=(_GSTEPS,),
        in_specs=[pl.BlockSpec((1, _GRB, dh), lambda i: (0, i, 0)),
                  pl.BlockSpec((1, _GRB, dh), lambda i: (1, i, 0)),
                  pl.BlockSpec((_GRB, dh), lambda i: (i, 0)),
                  pl.BlockSpec((_GRB, dh), lambda i: (i, 0)),
                  pl.BlockSpec((_GRB, 1), lambda i: (i, 0)),
                  pl.BlockSpec((1, d), lambda i: (0, 0)),
                  pl.BlockSpec((1, d), lambda i: (0, 0)),
                  pl.BlockSpec((1, d), lambda i: (0, 0))],
        out_specs=[pl.BlockSpec((_GRB, d), lambda i: (i, 0))],
        out_shape=[jax.ShapeDtypeStruct((N, d), jnp.float32)],
    )(agg, agg, h_lo, h_hi, deg_col, b, g, be)[0]


def _post3_body(ag0_ref, ag1_ref, hf_ref, deg_ref, x_ref, wsk_ref,
                b_ref, g_ref, be_ref, out_ref):
    dis = lax.rsqrt(deg_ref[...] + 1.0)
    full = ag0_ref[0] + ag1_ref[0] + hf_ref[...]
    skip = jnp.dot(x_ref[...], wsk_ref[...], preferred_element_type=jnp.float32)
    z = full * dis + b_ref[...] + skip
    mu = z.mean(-1, keepdims=True)
    var = ((z - mu) ** 2).mean(-1, keepdims=True)
    out_ref[...] = (z - mu) / jnp.sqrt(var + 1e-5) * g_ref[...] + be_ref[...]


def _tc_post3(agg, h_full, deg_col, x, wsk, b, g, be):
    d = h_full.shape[1]
    cin = x.shape[1]
    return pl.pallas_call(
        _post3_body,
        grid=(_GSTEPS,),
        in_specs=[pl.BlockSpec((1, _GRB, d), lambda i: (0, i, 0)),
                  pl.BlockSpec((1, _GRB, d), lambda i: (1, i, 0)),
                  pl.BlockSpec((_GRB, d), lambda i: (i, 0)),
                  pl.BlockSpec((_GRB, 1), lambda i: (i, 0)),
                  pl.BlockSpec((_GRB, cin), lambda i: (i, 0)),
                  pl.BlockSpec((cin, d), lambda i: (0, 0)),
                  pl.BlockSpec((1, d), lambda i: (0, 0)),
                  pl.BlockSpec((1, d), lambda i: (0, 0)),
                  pl.BlockSpec((1, d), lambda i: (0, 0))],
        out_specs=[pl.BlockSpec((_GRB, d), lambda i: (i, 0))],
        out_shape=[jax.ShapeDtypeStruct((N, d), jnp.float32)],
    )(agg, agg, h_full, deg_col, x, wsk, b, g, be)[0]


def _gather_body(ps_ref, pd_ref, hu_ref, hv_ref, hs_ref, ht_ref):
    hs_ref[...] = hu_ref[...]
    ht_ref[...] = hv_ref[...]


def _tc_gather_pairs(h3, ps, pd):
    d = h3.shape[2]
    grid_spec = pltpu.PrefetchScalarGridSpec(
        num_scalar_prefetch=2,
        grid=(P,),
        in_specs=[pl.BlockSpec((1, 1, d), lambda i, ps, pd: (ps[i], 0, 0)),
                  pl.BlockSpec((1, 1, d), lambda i, ps, pd: (pd[i], 0, 0))],
        out_specs=[pl.BlockSpec((1, 1, d), lambda i, ps, pd: (i, 0, 0)),
                   pl.BlockSpec((1, 1, d), lambda i, ps, pd: (i, 0, 0))],
    )
    return pl.pallas_call(
        _gather_body,
        grid_spec=grid_spec,
        out_shape=[jax.ShapeDtypeStruct((P, 1, d), jnp.float32),
                   jax.ShapeDtypeStruct((P, 1, d), jnp.float32)],
    )(ps, pd, h3, h3)


def _head_body(hs_ref, ht_ref, feats_ref, hits_ref, ep_ref, emb_ref,
               wq, wk, wv,
               cw1, cb1, cw2, cb2,
               aw1, ab1, aw2, ab2,
               jw1, jb1, jw2, jb2,
               mw1, mb1, mg, mbe, mw2, mb2, mw3, mb3,
               out_ref):
    hs = hs_ref[...]
    ht = ht_ref[...]
    feats = feats_ref[...]
    hits = hits_ref[...]
    cn = feats[:, 0:1]
    aa = feats[:, 1:2]
    du = feats[:, 2:3]
    dv = feats[:, 3:4]
    h1 = feats[:, 4:5]
    un = du + dv - cn
    ja = jnp.where(un > 0, cn / jnp.where(un > 0, un, 1.0), 0.0)

    ps = ep_ref[:, 0:1]
    pd = ep_ref[:, 1:2]
    spd = jnp.full_like(ps, MAX_SPD + 1)
    spd = jnp.where(hits[:, 2:3] > 0, 5, spd)
    spd = jnp.where(hits[:, 1:2] > 0, 4, spd)
    spd = jnp.where(hits[:, 0:1] > 0, 3, spd)
    spd = jnp.where(h1 > 0, 1, spd)
    spd = jnp.where(ps == pd, 0, spd)
    emb = emb_ref[...]
    bspd = jnp.zeros_like(cn)
    for k in range(MAX_SPD + 2):
        bspd = jnp.where(spd == k, emb[0, k], bspd)

    q = jnp.dot(hs, wq[...], preferred_element_type=jnp.float32)
    kk = jnp.dot(ht, wk[...], preferred_element_type=jnp.float32)
    v = jnp.dot(ht, wv[...], preferred_element_type=jnp.float32)
    a = (q * kk).sum(-1, keepdims=True) * (1.0 / (OUT ** 0.5))

    def mlp(val, w1, b1, w2, b2):
        h = jnp.maximum(jnp.dot(val, w1[...], preferred_element_type=jnp.float32)
                        + b1[...], 0.0)
        return jnp.dot(h, w2[...], preferred_element_type=jnp.float32) + b2[...]

    s = (bspd
         + mlp(cn, cw1, cb1, cw2, cb2)
         + mlp(aa, aw1, ab1, aw2, ab2)
         + mlp(ja, jw1, jb1, jw2, jb2))
    attn = jax.nn.sigmoid(a + s) * v
    z = jnp.concatenate([hs, ht, attn], axis=-1)
    z = jnp.dot(z, mw1[...], preferred_element_type=jnp.float32) + mb1[...]
    mu = z.mean(-1, keepdims=True)
    var = ((z - mu) ** 2).mean(-1, keepdims=True)
    z = (z - mu) / jnp.sqrt(var + 1e-5) * mg[...] + mbe[...]
    z = jnp.maximum(z, 0.0)
    z = jnp.maximum(jnp.dot(z, mw2[...], preferred_element_type=jnp.float32)
                    + mb2[...], 0.0)
    z = jnp.dot(z, mw3[...], preferred_element_type=jnp.float32) + mb3[...]
    out_ref[...] = z


def _head(hs, ht, feats, hits, ep, p):
    emb_row = jnp.zeros((1, 8), jnp.float32).at[0, :MAX_SPD + 2].set(
        p['spd_emb'][:, 0])
    args = (hs, ht, feats, hits, ep, emb_row,
            p['WQ'], p['WK'], p['WV'],
            p['cn_w1'], p['cn_b1'][None, :], p['cn_w2'], p['cn_b2'][None, :],
            p['aa_w1'], p['aa_b1'][None, :], p['aa_w2'], p['aa_b2'][None, :],
            p['ja_w1'], p['ja_b1'][None, :], p['ja_w2'], p['ja_b2'][None, :],
            p['m_w1'], p['m_b1'][None, :], p['m_g'][None, :], p['m_be'][None, :],
            p['m_w2'], p['m_b2'][None, :], p['m_w3'], p['m_b3'][None, :])
    out = pl.pallas_call(
        _head_body,
        out_shape=jax.ShapeDtypeStruct((P, 1), jnp.float32),
    )(*args)
    return out[:, 0]


def kernel(x, edge_index, edge_pairs, params):
    p = params
    ei = edge_index.astype(jnp.int32)
    esrc = ei[0]
    edst = ei[1]
    ep = edge_pairs.astype(jnp.int32)
    ps = ep[:, 0]
    pd = ep[:, 1]

    adj_flat, deg_hist = _sc_build(esrc, edst)
    adj = adj_flat.reshape(NP, NP)
    deg_col = deg_hist[:N].reshape(N, 1)

    adj_bf, _, ldi = _tc_prep(adj)
    adj3 = adj_bf.reshape(NP, 1, NP)
    r1_3, feats3 = _tc_pair(adj3, ldi, ps, pd)
    r1 = r1_3.reshape(P, NP)
    r2 = _tc_spd_step(r1, adj_bf)
    r3 = _tc_spd_step(r2, adj_bf)
    r4 = _tc_spd_step(r3, adj_bf)
    hits = _tc_extract(r2, r3, r4, ps, pd).reshape(P, 8)
    feats = feats3.reshape(P, 8)

    lo, hi = _tc_scale(x, p['W1'], deg_col)
    agg = _sc_aggregate(esrc, edst, lo, hi)
    h = _tc_post(agg, lo, hi, deg_col, p['b1'][None, :], p['g1'][None, :],
                 p['be1'][None, :], relu=True)
    lo, hi = _tc_scale(h, p['W2'], deg_col)
    agg = _sc_aggregate(esrc, edst, lo, hi)
    h = _tc_post(agg, lo, hi, deg_col, p['b2'][None, :], p['g2'][None, :],
                 p['be2'][None, :], relu=True)
    hf = _tc_scale(h, p['W3'], deg_col, split=False)[0]
    agg = _sc_aggregate_edges(esrc, edst, hf)
    h = _tc_post3(agg, hf, deg_col, x, p['Wskip'], p['b3'][None, :],
                  p['g3'][None, :], p['be3'][None, :])

    hs3, ht3 = _tc_gather_pairs(h.reshape(N, 1, OUT), ps, pd)
    return _head(hs3.reshape(P, OUT), ht3.reshape(P, OUT), feats, hits, ep, p)
